# baseline (speedup 1.0000x reference)
import functools

import jax
import jax.numpy as jnp
from jax.experimental import pallas as pl
from jax.experimental.pallas import tpu as pltpu

_B, _C, _H, _W = 64, 768, 24, 24
_HW = _H * _W
_HIDDEN = 192
_NE = 8
_ROWS = 16
_HWH = _HW // 2


def _router_kernel(xa_ref, xb_ref, w1t_ref, b1_ref, w2t_ref, b2_ref,
                   idx_ref, wgt_ref, pooled_ref):
    b = pl.program_id(0)
    s = jnp.sum(xa_ref[:, :, :], axis=1) + jnp.sum(xb_ref[:, :, :], axis=1)
    pooled_ref[pl.ds(b * _ROWS, _ROWS), :] = s * (1.0 / _HW)

    @pl.when(b == _B // _ROWS - 1)
    def _finalize():
        pooled = pooled_ref[:, :]
        h = jax.lax.dot_general(
            pooled, w1t_ref[:, :], (((1,), (1,)), ((), ())),
            preferred_element_type=jnp.float32) + b1_ref[0]
        h = 0.5 * h * (1.0 + jax.lax.erf(h * (2.0 ** -0.5)))
        logits = jax.lax.dot_general(
            h, w2t_ref[:, :], (((1,), (1,)), ((), ())),
            preferred_element_type=jnp.float32) + b2_ref[0]

        eidx = jax.lax.broadcasted_iota(jnp.int32, (_B, _NE), 1)
        m1 = jnp.max(logits, axis=-1, keepdims=True)
        i1 = jnp.min(jnp.where(logits == m1, eidx, _NE), axis=-1, keepdims=True)
        masked = jnp.where(eidx == i1, -jnp.inf, logits)
        m2 = jnp.max(masked, axis=-1, keepdims=True)
        i2 = jnp.min(jnp.where(masked == m2, eidx, _NE), axis=-1, keepdims=True)

        e2 = jnp.exp(m2 - m1)
        denom = 1.0 + e2
        idx_ref[:, :] = jnp.concatenate([i1, i2], axis=1)
        wgt_ref[:, :] = jnp.concatenate([1.0 / denom, e2 / denom], axis=1)


@functools.partial(jax.jit, static_argnames=())
def kernel(x, W1, b1, W2, b2):
    xt = jnp.transpose(x, (0, 2, 3, 1)).reshape(_B, _HW, _C)
    idx, wgt = pl.pallas_call(
        _router_kernel,
        grid=(_B // _ROWS,),
        in_specs=[
            pl.BlockSpec((_ROWS, _HWH, _C), lambda b: (b, 0, 0)),
            pl.BlockSpec((_ROWS, _HWH, _C), lambda b: (b, 1, 0)),
            pl.BlockSpec((_HIDDEN, _C), lambda b: (0, 0)),
            pl.BlockSpec((1, _HIDDEN), lambda b: (0, 0)),
            pl.BlockSpec((_NE, _HIDDEN), lambda b: (0, 0)),
            pl.BlockSpec((1, _NE), lambda b: (0, 0)),
        ],
        out_specs=[
            pl.BlockSpec((_B, 2), lambda b: (0, 0)),
            pl.BlockSpec((_B, 2), lambda b: (0, 0)),
        ],
        out_shape=[
            jax.ShapeDtypeStruct((_B, 2), jnp.int32),
            jax.ShapeDtypeStruct((_B, 2), jnp.float32),
        ],
        scratch_shapes=[pltpu.VMEM((_B, _C), jnp.float32)],
    )(xt, xt, W1.T, b1.reshape(1, _HIDDEN), W2.T, b2.reshape(1, _NE))
    return idx, wgt


# speedup vs baseline: 1.0691x; 1.0691x over previous
import functools

import jax
import jax.numpy as jnp
from jax.experimental import pallas as pl
from jax.experimental.pallas import tpu as pltpu

_B, _C, _H, _W = 64, 768, 24, 24
_HW = _H * _W
_HIDDEN = 192
_NE = 8
_ROWS = 8
_HWH = _HW // 2


def _router_kernel(xa_ref, xb_ref, w1t_ref, b1_ref, w2t_ref, b2_ref,
                   idx_ref, wgt_ref, pooled_ref):
    b = pl.program_id(0)
    s = jnp.sum(xa_ref[:, :, :], axis=1) + jnp.sum(xb_ref[:, :, :], axis=1)
    pooled_ref[pl.ds(b * _ROWS, _ROWS), :] = s * (1.0 / _HW)

    @pl.when(b == _B // _ROWS - 1)
    def _finalize():
        pooled = pooled_ref[:, :]
        h = jax.lax.dot_general(
            pooled, w1t_ref[:, :], (((1,), (1,)), ((), ())),
            preferred_element_type=jnp.float32) + b1_ref[0]
        h = 0.5 * h * (1.0 + jax.lax.erf(h * (2.0 ** -0.5)))
        logits = jax.lax.dot_general(
            h, w2t_ref[:, :], (((1,), (1,)), ((), ())),
            preferred_element_type=jnp.float32) + b2_ref[0]

        eidx = jax.lax.broadcasted_iota(jnp.int32, (_B, _NE), 1)
        m1 = jnp.max(logits, axis=-1, keepdims=True)
        i1 = jnp.min(jnp.where(logits == m1, eidx, _NE), axis=-1, keepdims=True)
        masked = jnp.where(eidx == i1, -jnp.inf, logits)
        m2 = jnp.max(masked, axis=-1, keepdims=True)
        i2 = jnp.min(jnp.where(masked == m2, eidx, _NE), axis=-1, keepdims=True)

        e2 = jnp.exp(m2 - m1)
        denom = 1.0 + e2
        idx_ref[:, :] = jnp.concatenate([i1, i2], axis=1)
        wgt_ref[:, :] = jnp.concatenate([1.0 / denom, e2 / denom], axis=1)


@functools.partial(jax.jit, static_argnames=())
def kernel(x, W1, b1, W2, b2):
    xt = jnp.transpose(x, (0, 2, 3, 1)).reshape(_B, _HW, _C)
    idx, wgt = pl.pallas_call(
        _router_kernel,
        grid=(_B // _ROWS,),
        in_specs=[
            pl.BlockSpec((_ROWS, _HWH, _C), lambda b: (b, 0, 0)),
            pl.BlockSpec((_ROWS, _HWH, _C), lambda b: (b, 1, 0)),
            pl.BlockSpec((_HIDDEN, _C), lambda b: (0, 0)),
            pl.BlockSpec((1, _HIDDEN), lambda b: (0, 0)),
            pl.BlockSpec((_NE, _HIDDEN), lambda b: (0, 0)),
            pl.BlockSpec((1, _NE), lambda b: (0, 0)),
        ],
        out_specs=[
            pl.BlockSpec((_B, 2), lambda b: (0, 0)),
            pl.BlockSpec((_B, 2), lambda b: (0, 0)),
        ],
        out_shape=[
            jax.ShapeDtypeStruct((_B, 2), jnp.int32),
            jax.ShapeDtypeStruct((_B, 2), jnp.float32),
        ],
        scratch_shapes=[pltpu.VMEM((_B, _C), jnp.float32)],
    )(xt, xt, W1.T, b1.reshape(1, _HIDDEN), W2.T, b2.reshape(1, _NE))
    return idx, wgt


# speedup vs baseline: 1.1118x; 1.0399x over previous
import functools

import jax
import jax.numpy as jnp
from jax.experimental import pallas as pl
from jax.experimental.pallas import tpu as pltpu

_B, _C, _H, _W = 64, 768, 24, 24
_HW = _H * _W
_HIDDEN = 192
_NE = 8
_ROWS = 8
_HWQ = _HW // 4


def _router_kernel(xa_ref, xb_ref, xc_ref, xd_ref, w1t_ref, b1_ref,
                   w2t_ref, b2_ref, idx_ref, wgt_ref, pooled_ref):
    b = pl.program_id(0)
    s = ((jnp.sum(xa_ref[:, :, :], axis=1) + jnp.sum(xb_ref[:, :, :], axis=1))
         + (jnp.sum(xc_ref[:, :, :], axis=1) + jnp.sum(xd_ref[:, :, :], axis=1)))
    pooled_ref[pl.ds(b * _ROWS, _ROWS), :] = s * (1.0 / _HW)

    @pl.when(b == _B // _ROWS - 1)
    def _finalize():
        pooled = pooled_ref[:, :]
        h = jax.lax.dot_general(
            pooled, w1t_ref[:, :], (((1,), (1,)), ((), ())),
            preferred_element_type=jnp.float32) + b1_ref[0]
        h = 0.5 * h * (1.0 + jax.lax.erf(h * (2.0 ** -0.5)))
        logits = jax.lax.dot_general(
            h, w2t_ref[:, :], (((1,), (1,)), ((), ())),
            preferred_element_type=jnp.float32) + b2_ref[0]

        eidx = jax.lax.broadcasted_iota(jnp.int32, (_B, _NE), 1)
        m1 = jnp.max(logits, axis=-1, keepdims=True)
        i1 = jnp.min(jnp.where(logits == m1, eidx, _NE), axis=-1, keepdims=True)
        masked = jnp.where(eidx == i1, -jnp.inf, logits)
        m2 = jnp.max(masked, axis=-1, keepdims=True)
        i2 = jnp.min(jnp.where(masked == m2, eidx, _NE), axis=-1, keepdims=True)

        e2 = jnp.exp(m2 - m1)
        denom = 1.0 + e2
        idx_ref[:, :] = jnp.concatenate([i1, i2], axis=1)
        wgt_ref[:, :] = jnp.concatenate([1.0 / denom, e2 / denom], axis=1)


@functools.partial(jax.jit, static_argnames=())
def kernel(x, W1, b1, W2, b2):
    xt = jnp.transpose(x, (0, 2, 3, 1)).reshape(_B, _HW, _C)
    idx, wgt = pl.pallas_call(
        _router_kernel,
        grid=(_B // _ROWS,),
        in_specs=[
            pl.BlockSpec((_ROWS, _HWQ, _C), lambda b: (b, 0, 0)),
            pl.BlockSpec((_ROWS, _HWQ, _C), lambda b: (b, 1, 0)),
            pl.BlockSpec((_ROWS, _HWQ, _C), lambda b: (b, 2, 0)),
            pl.BlockSpec((_ROWS, _HWQ, _C), lambda b: (b, 3, 0)),
            pl.BlockSpec((_HIDDEN, _C), lambda b: (0, 0)),
            pl.BlockSpec((1, _HIDDEN), lambda b: (0, 0)),
            pl.BlockSpec((_NE, _HIDDEN), lambda b: (0, 0)),
            pl.BlockSpec((1, _NE), lambda b: (0, 0)),
        ],
        out_specs=[
            pl.BlockSpec((_B, 2), lambda b: (0, 0)),
            pl.BlockSpec((_B, 2), lambda b: (0, 0)),
        ],
        out_shape=[
            jax.ShapeDtypeStruct((_B, 2), jnp.int32),
            jax.ShapeDtypeStruct((_B, 2), jnp.float32),
        ],
        scratch_shapes=[pltpu.VMEM((_B, _C), jnp.float32)],
    )(xt, xt, xt, xt, W1.T, b1.reshape(1, _HIDDEN), W2.T, b2.reshape(1, _NE))
    return idx, wgt
